# baseline (speedup 1.0000x reference)
import functools

import jax
import jax.numpy as jnp
from jax import lax
from jax.experimental import pallas as pl
from jax.experimental.pallas import tpu as pltpu
from jax.experimental.pallas import tpu_sc as plsc

_NUM_WORKERS = 32
_IDX_MINOR = 128
_CHUNK = 512
_LANES = 16


def _embed_body(max_len, per_worker, idx_hbm, table_hbm, pos_hbm, out_hbm,
                idx_v, rows_v, pos_v, gsem):
    d = pos_v.shape[1]
    n_dsl = d // _LANES
    gpb = _CHUNK // _IDX_MINOR
    wid = lax.axis_index("s") * 2 + lax.axis_index("c")
    base = wid * per_worker
    idx_row_base = wid * (per_worker // _IDX_MINOR)

    pltpu.sync_copy(pos_hbm, pos_v)

    n_chunks = per_worker // _CHUNK
    scale = jnp.float32(float(d) ** 0.5)

    def chunk_body(c, _):
        cb = base + c * _CHUNK
        pltpu.sync_copy(idx_hbm.at[pl.ds(idx_row_base + c * gpb, gpb)], idx_v)
        copies = [
            pltpu.async_copy(table_hbm.at[idx_v.at[j]],
                             rows_v.at[pl.ds(j * _IDX_MINOR, _IDX_MINOR)],
                             gsem)
            for j in range(gpb)
        ]
        for cp in copies:
            cp.wait()

        pr0 = lax.rem(jnp.int32(c) * _CHUNK, max_len)

        def row_body(r, pr):
            for k in range(n_dsl):
                sl = pl.ds(k * _LANES, _LANES)
                pv = pos_v[pr, sl]
                rows_v[r, sl] = rows_v[r, sl] * scale + pv
            return jnp.where(pr == max_len - 1, 0, pr + 1)

        lax.fori_loop(0, _CHUNK, row_body, pr0)
        pltpu.sync_copy(rows_v, out_hbm.at[pl.ds(cb, _CHUNK)])
        return 0

    lax.fori_loop(0, n_chunks, chunk_body, 0)


def kernel(item_id, item_table, pos_table):
    batch, max_len = item_id.shape
    d = item_table.shape[1]
    n_flat = batch * max_len
    per_worker = n_flat // _NUM_WORKERS

    idx2d = item_id.reshape(n_flat // _IDX_MINOR, _IDX_MINOR)

    mesh = plsc.VectorSubcoreMesh(core_axis_name="c", subcore_axis_name="s")
    body = functools.partial(_embed_body, max_len, per_worker)
    out = pl.kernel(
        body,
        out_type=jax.ShapeDtypeStruct((n_flat, d), jnp.float32),
        scratch_types=[
            pltpu.VMEM((_CHUNK // _IDX_MINOR, _IDX_MINOR), jnp.int32),
            pltpu.VMEM((_CHUNK, d), jnp.float32),
            pltpu.VMEM((max_len, d), jnp.float32),
            pltpu.SemaphoreType.DMA,
        ],
        mesh=mesh,
        compiler_params=pltpu.CompilerParams(use_tc_tiling_on_sc=False),
    )(idx2d, item_table, pos_table)
    return out.reshape(batch, max_len, d)


# speedup vs baseline: 1.0698x; 1.0698x over previous
import functools

import jax
import jax.numpy as jnp
from jax import lax
from jax.experimental import pallas as pl
from jax.experimental.pallas import tpu as pltpu
from jax.experimental.pallas import tpu_sc as plsc

_NUM_WORKERS = 32
_IDX_MINOR = 128
_CHUNK = 512
_LANES = 16


def _embed_body(max_len, per_worker, idx_hbm, table_hbm, pos_hbm, out_hbm,
                idx_all, rows0, rows1, pos_v,
                gsem0, gsem1, osem0, osem1):
    d = pos_v.shape[1]
    n_dsl = d // _LANES
    gpb = _CHUNK // _IDX_MINOR
    idx_rows = per_worker // _IDX_MINOR
    n_chunks = per_worker // _CHUNK
    wid = lax.axis_index("s") * 2 + lax.axis_index("c")
    base = wid * per_worker
    scale = jnp.float32(float(d) ** 0.5)

    bufs = (rows0, rows1)
    gsems = (gsem0, gsem1)
    osems = (osem0, osem1)

    pltpu.sync_copy(idx_hbm.at[pl.ds(wid * idx_rows, idx_rows)], idx_all)
    pltpu.sync_copy(pos_hbm, pos_v)

    def prime(c, buf):
        for j in range(gpb):
            pltpu.async_copy(
                table_hbm.at[idx_all.at[c * gpb + j]],
                bufs[buf].at[pl.ds(j * _IDX_MINOR, _IDX_MINOR)],
                gsems[buf])

    def drain_out(buf):
        pltpu.make_async_copy(
            bufs[buf], out_hbm.at[pl.ds(0, _CHUNK)], osems[buf]).wait()

    def wait_gather(buf):
        pltpu.make_async_copy(
            out_hbm.at[pl.ds(0, _CHUNK)], bufs[buf], gsems[buf]).wait()

    prime(0, 0)

    def pair_body(i, _):
        for b in range(2):
            c = 2 * i + b
            nxt = c + 1

            @pl.when(jnp.logical_and(nxt < n_chunks, c >= 1))
            def _():
                drain_out(1 - b)

            @pl.when(nxt < n_chunks)
            def _():
                prime(nxt, 1 - b)

            wait_gather(b)

            pr0 = lax.rem(c * _CHUNK, max_len)
            for j in range(_CHUNK // max_len + 2):
                lo = jnp.clip(j * max_len - pr0, 0, _CHUNK)
                hi = jnp.clip((j + 1) * max_len - pr0, 0, _CHUNK)
                shift = pr0 - j * max_len

                def span_body(r, _, b=b, shift=shift):
                    for k in range(n_dsl):
                        sl = pl.ds(k * _LANES, _LANES)
                        pv = pos_v[r + shift, sl]
                        bufs[b][r, sl] = bufs[b][r, sl] * scale + pv
                    return 0

                lax.fori_loop(lo, hi, span_body, 0)

            pltpu.async_copy(
                bufs[b], out_hbm.at[pl.ds(base + c * _CHUNK, _CHUNK)],
                osems[b])
        return 0

    lax.fori_loop(0, n_chunks // 2, pair_body, 0)
    for b in range(2):
        drain_out(b)


def kernel(item_id, item_table, pos_table):
    batch, max_len = item_id.shape
    d = item_table.shape[1]
    n_flat = batch * max_len
    per_worker = n_flat // _NUM_WORKERS

    idx2d = item_id.reshape(n_flat // _IDX_MINOR, _IDX_MINOR)

    mesh = plsc.VectorSubcoreMesh(core_axis_name="c", subcore_axis_name="s")
    body = functools.partial(_embed_body, max_len, per_worker)
    out = pl.kernel(
        body,
        out_type=jax.ShapeDtypeStruct((n_flat, d), jnp.float32),
        scratch_types=[
            pltpu.VMEM((per_worker // _IDX_MINOR, _IDX_MINOR), jnp.int32),
            pltpu.VMEM((_CHUNK, d), jnp.float32),
            pltpu.VMEM((_CHUNK, d), jnp.float32),
            pltpu.VMEM((max_len, d), jnp.float32),
            pltpu.SemaphoreType.DMA,
            pltpu.SemaphoreType.DMA,
            pltpu.SemaphoreType.DMA,
            pltpu.SemaphoreType.DMA,
        ],
        mesh=mesh,
        compiler_params=pltpu.CompilerParams(use_tc_tiling_on_sc=False),
    )(idx2d, item_table, pos_table)
    return out.reshape(batch, max_len, d)


# speedup vs baseline: 1.4105x; 1.3185x over previous
import functools

import jax
import jax.numpy as jnp
from jax import lax
from jax.experimental import pallas as pl
from jax.experimental.pallas import tpu as pltpu
from jax.experimental.pallas import tpu_sc as plsc

_NUM_WORKERS = 32
_IDX_MINOR = 100
_CHUNK = 400
_LANES = 16
_UNROLL = 8


def _embed_body(max_len, per_worker, idx_hbm, table_hbm, pos_hbm, out_hbm,
                idx_all, rows0, rows1, pos_v,
                gsem0, gsem1, osem0, osem1):
    d = pos_v.shape[1]
    n_dsl = d // _LANES
    gpb = _CHUNK // _IDX_MINOR
    idx_rows = per_worker // _IDX_MINOR
    n_chunks = per_worker // _CHUNK
    reps = _CHUNK // max_len
    wid = lax.axis_index("s") * 2 + lax.axis_index("c")
    base = wid * per_worker
    scale = jnp.float32(float(d) ** 0.5)

    bufs = (rows0, rows1)
    gsems = (gsem0, gsem1)
    osems = (osem0, osem1)

    pltpu.sync_copy(idx_hbm.at[pl.ds(wid * idx_rows, idx_rows)], idx_all)
    pltpu.sync_copy(pos_hbm, pos_v)

    def prime(c, buf):
        for j in range(gpb):
            pltpu.async_copy(
                table_hbm.at[idx_all.at[c * gpb + j]],
                bufs[buf].at[pl.ds(j * _IDX_MINOR, _IDX_MINOR)],
                gsems[buf])

    def drain_out(buf):
        pltpu.make_async_copy(
            bufs[buf], out_hbm.at[pl.ds(0, _CHUNK)], osems[buf]).wait()

    def wait_gather(buf):
        pltpu.make_async_copy(
            out_hbm.at[pl.ds(0, _CHUNK)], bufs[buf], gsems[buf]).wait()

    def compute(buf):
        @plsc.parallel_loop(0, max_len, unroll=_UNROLL)
        def _(r):
            for k in range(n_dsl):
                sl = pl.ds(k * _LANES, _LANES)
                pv = pos_v[r, sl]
                for t in range(reps):
                    row = t * max_len + r
                    bufs[buf][row, sl] = bufs[buf][row, sl] * scale + pv

    prime(0, 0)

    def pair_body(i, _):
        for b in range(2):
            c = 2 * i + b
            nxt = c + 1

            @pl.when(jnp.logical_and(nxt < n_chunks, c >= 1))
            def _():
                drain_out(1 - b)

            @pl.when(nxt < n_chunks)
            def _():
                prime(nxt, 1 - b)

            wait_gather(b)
            compute(b)
            pltpu.async_copy(
                bufs[b], out_hbm.at[pl.ds(base + c * _CHUNK, _CHUNK)],
                osems[b])
        return 0

    lax.fori_loop(0, n_chunks // 2, pair_body, 0)
    for b in range(2):
        drain_out(b)


def kernel(item_id, item_table, pos_table):
    batch, max_len = item_id.shape
    d = item_table.shape[1]
    n_flat = batch * max_len
    per_worker = n_flat // _NUM_WORKERS

    idx2d = item_id.reshape(n_flat // _IDX_MINOR, _IDX_MINOR)

    mesh = plsc.VectorSubcoreMesh(core_axis_name="c", subcore_axis_name="s")
    body = functools.partial(_embed_body, max_len, per_worker)
    out = pl.kernel(
        body,
        out_type=jax.ShapeDtypeStruct((n_flat, d), jnp.float32),
        scratch_types=[
            pltpu.VMEM((per_worker // _IDX_MINOR, _IDX_MINOR), jnp.int32),
            pltpu.VMEM((_CHUNK, d), jnp.float32),
            pltpu.VMEM((_CHUNK, d), jnp.float32),
            pltpu.VMEM((max_len, d), jnp.float32),
            pltpu.SemaphoreType.DMA,
            pltpu.SemaphoreType.DMA,
            pltpu.SemaphoreType.DMA,
            pltpu.SemaphoreType.DMA,
        ],
        mesh=mesh,
        compiler_params=pltpu.CompilerParams(use_tc_tiling_on_sc=False),
    )(idx2d, item_table, pos_table)
    return out.reshape(batch, max_len, d)
